# baseline (speedup 1.0000x reference)
import functools

import jax
import jax.numpy as jnp
from jax import lax
from jax.experimental import pallas as pl
from jax.experimental.pallas import tpu as pltpu
from jax.experimental.pallas import tpu_sc as plsc

N = 10000
NPAD = 10112
E = 320000
DIN = 128
DOUT = 128
CHUNK = 128
EDGES_PER_CORE = E // 2
NUM_CHUNKS = EDGES_PER_CORE // CHUNK
NS = 16
ROWS_PER_TILE = NPAD // NS
ZB = (128, 128, 128, 128, 120)
CHUNKS_PER_TILE = -(-NUM_CHUNKS // NS)


def _mm_body(x_ref, w_ref, o_ref):
    o_ref[...] = jnp.dot(x_ref[...], w_ref[...], preferred_element_type=jnp.float32)


def _matmul(x, W):
    bm = 10000
    return pl.pallas_call(
        _mm_body,
        grid=(N // bm,),
        in_specs=[
            pl.BlockSpec((bm, DIN), lambda i: (i, 0)),
            pl.BlockSpec((DIN, DOUT), lambda i: (0, 0)),
        ],
        out_specs=pl.BlockSpec((bm, DOUT), lambda i: (i, 0)),
        out_shape=jax.ShapeDtypeStruct((N, DOUT), jnp.float32),
    )(x, W)


def _combine_body(p_ref, o_ref):
    o_ref[...] = jnp.maximum(p_ref[0] + p_ref[1], 0.0)


def _combine_relu(partials):
    bm = 10000
    return pl.pallas_call(
        _combine_body,
        grid=(N // bm,),
        in_specs=[pl.BlockSpec((2, bm, DOUT), lambda i: (0, i, 0))],
        out_specs=pl.BlockSpec((bm, DOUT), lambda i: (i, 0)),
        out_shape=jax.ShapeDtypeStruct((N, DOUT), jnp.float32),
    )(partials)


@functools.partial(
    pl.kernel,
    out_type=jax.ShapeDtypeStruct((2, NPAD, DOUT), jnp.float32),
    mesh=plsc.VectorSubcoreMesh(core_axis_name="c", subcore_axis_name="s"),
    scratch_types=[
        pltpu.VMEM((CHUNK,), jnp.int32),
        pltpu.VMEM((CHUNK,), jnp.int32),
        pltpu.VMEM((CHUNK,), jnp.int32),
        pltpu.VMEM((CHUNK,), jnp.int32),
        pltpu.VMEM((CHUNK,), jnp.int32),
        pltpu.VMEM((CHUNK,), jnp.float32),
        pltpu.VMEM((CHUNK,), jnp.float32),
        pltpu.VMEM((CHUNK, DOUT), jnp.float32),
        pltpu.VMEM((CHUNK, DOUT), jnp.float32),
        pltpu.VMEM((CHUNK, DOUT), jnp.float32),
        pltpu.VMEM_SHARED((NPAD, DOUT), jnp.float32),
        pltpu.SemaphoreType.DMA,
        pltpu.SemaphoreType.DMA,
        pltpu.SemaphoreType.DMA,
        pltpu.SemaphoreType.DMA,
        pltpu.SemaphoreType.DMA,
        pltpu.SemaphoreType.DMA,
        pltpu.SemaphoreType.DMA,
        pltpu.SemaphoreType.DMA,
    ],
)
def _sc_aggregate(pre_hbm, src_hbm, dst_hbm, ew_hbm, out_hbm,
                  src_0, src_1, src_2, dst_0, dst_1,
                  ew_0, ew_1, rows_0, rows_1, rows_2, acc,
                  sem_s0, sem_s1, sem_s2, sem_d0, sem_d1,
                  sem_g0, sem_g1, sem_g2):
    c = lax.axis_index("c")
    s = lax.axis_index("s")
    row0 = s * ROWS_PER_TILE
    SRC = (src_0, src_1, src_2)
    DST = (dst_0, dst_1)
    EW = (ew_0, ew_1)
    ROWS = (rows_0, rows_1, rows_2)
    SSEM = (sem_s0, sem_s1, sem_s2)
    DSEM = (sem_d0, sem_d1)
    GSEM = (sem_g0, sem_g1, sem_g2)

    def _e0(i):
        return c * EDGES_PER_CORE + (s + i * NS) * CHUNK

    def _src_cp(i, k3):
        return pltpu.make_async_copy(
            src_hbm.at[pl.ds(_e0(i), CHUNK)], SRC[k3], SSEM[k3])

    def _de_cps(i, k2):
        return (
            pltpu.make_async_copy(
                dst_hbm.at[pl.ds(_e0(i), CHUNK)], DST[k2], DSEM[k2]),
            pltpu.make_async_copy(
                ew_hbm.at[pl.ds(_e0(i), CHUNK)], EW[k2], DSEM[k2]),
        )

    def _start_src(i, k3):
        @pl.when(s + i * NS < NUM_CHUNKS)
        def _():
            _src_cp(i, k3).start()

    def _start_de(i, k2):
        @pl.when(s + i * NS < NUM_CHUNKS)
        def _():
            for cp in _de_cps(i, k2):
                cp.start()

    def _gth(k):
        return pltpu.make_async_copy(pre_hbm.at[SRC[k]], ROWS[k], GSEM[k])

    def _scale(k3, k2):
        wv, rv = EW[k2], ROWS[k3]

        def body(eg, carry2):
            w16 = wv[pl.ds(eg * 16, 16)]
            for kk in range(16):
                e = eg * 16 + kk
                wk = w16[kk]
                for j in range(DOUT // 16):
                    sl = pl.ds(j * 16, 16)
                    rv[e, sl] = rv[e, sl] * wk
            return carry2

        lax.fori_loop(0, CHUNK // 16, body, 0)

    def _zero_row(r, carry):
        for j in range(DOUT // 16):
            rows_0[r, pl.ds(j * 16, 16)] = jnp.zeros((16,), jnp.float32)
        return carry

    lax.fori_loop(0, 128, _zero_row, 0)
    off = 0
    for zb in ZB:
        pltpu.sync_copy(rows_0.at[pl.ds(0, zb)],
                        acc.at[pl.ds(row0 + off, zb)])
        off += zb
    plsc.subcore_barrier()

    for k in range(3):
        _start_src(k, k)
    for k in range(2):
        _start_de(k, k)
    for k in range(2):
        @pl.when(s + k * NS < NUM_CHUNKS)
        def _(k=k):
            _src_cp(k, k).wait()
            _gth(k).start()

    def _slot(i, k3, k2):
        n3 = (k3 + 2) % 3

        @pl.when(s + i * NS < NUM_CHUNKS)
        def _():
            _gth(k3).wait()

            @pl.when(s + (i + 2) * NS < NUM_CHUNKS)
            def _():
                _src_cp(i + 2, n3).wait()
                _gth(n3).start()

            for cp in _de_cps(i, k2):
                cp.wait()
            _scale(k3, k2)
            pltpu.sync_copy(ROWS[k3], acc.at[DST[k2]], add=True)
            _start_src(i + 3, k3)
            _start_de(i + 2, k2)

    def _six(t, carry):
        for w in range(6):
            _slot(6 * t + w, w % 3, w % 2)
        return carry

    lax.fori_loop(0, (CHUNKS_PER_TILE + 6) // 6, _six, 0)
    plsc.subcore_barrier()

    pltpu.sync_copy(acc.at[pl.ds(row0, ROWS_PER_TILE)],
                    out_hbm.at[c, pl.ds(row0, ROWS_PER_TILE)])


def kernel(x, edge_index, edge_weight, W):
    pre = _matmul(x, W)
    partials = _sc_aggregate(pre, edge_index[0], edge_index[1], edge_weight)
    return _combine_relu(partials)


# speedup vs baseline: 1.0026x; 1.0026x over previous
import functools

import jax
import jax.numpy as jnp
from jax import lax
from jax.experimental import pallas as pl
from jax.experimental.pallas import tpu as pltpu
from jax.experimental.pallas import tpu_sc as plsc

N = 10000
NPAD = 10240
E = 320000
DIN = 128
DOUT = 128
CHUNK = 128
EDGES_PER_CORE = E // 2
NUM_CHUNKS = EDGES_PER_CORE // CHUNK
NS = 16
ROWS_PER_TILE = NPAD // NS
RB = 128
CHUNKS_PER_TILE = -(-NUM_CHUNKS // NS)


def _mm_body(x_ref, w_ref, o_ref):
    o_ref[...] = jnp.dot(x_ref[...], w_ref[...], preferred_element_type=jnp.float32)


def _matmul(x, W):
    bm = 10000
    return pl.pallas_call(
        _mm_body,
        grid=(N // bm,),
        in_specs=[
            pl.BlockSpec((bm, DIN), lambda i: (i, 0)),
            pl.BlockSpec((DIN, DOUT), lambda i: (0, 0)),
        ],
        out_specs=pl.BlockSpec((bm, DOUT), lambda i: (i, 0)),
        out_shape=jax.ShapeDtypeStruct((N, DOUT), jnp.float32),
    )(x, W)


def _combine_body(p_ref, o_ref):
    o_ref[...] = jnp.maximum(p_ref[0] + p_ref[1], 0.0)


def _combine_relu(partials):
    bm = 10000
    return pl.pallas_call(
        _combine_body,
        grid=(N // bm,),
        in_specs=[pl.BlockSpec((2, bm, DOUT), lambda i: (0, i, 0))],
        out_specs=pl.BlockSpec((bm, DOUT), lambda i: (i, 0)),
        out_shape=jax.ShapeDtypeStruct((N, DOUT), jnp.float32),
    )(partials)


@functools.partial(
    pl.kernel,
    out_type=jax.ShapeDtypeStruct((2, NPAD, DOUT), jnp.float32),
    mesh=plsc.VectorSubcoreMesh(core_axis_name="c", subcore_axis_name="s"),
    scratch_types=[
        pltpu.VMEM((CHUNK,), jnp.int32),
        pltpu.VMEM((CHUNK,), jnp.int32),
        pltpu.VMEM((CHUNK,), jnp.float32),
        pltpu.VMEM((CHUNK,), jnp.int32),
        pltpu.VMEM((CHUNK,), jnp.int32),
        pltpu.VMEM((CHUNK,), jnp.float32),
        pltpu.VMEM((CHUNK, DOUT), jnp.float32),
        pltpu.VMEM((CHUNK, DOUT), jnp.float32),
        pltpu.VMEM_SHARED((NPAD, DOUT), jnp.float32),
        pltpu.SemaphoreType.DMA,
        pltpu.SemaphoreType.DMA,
        pltpu.SemaphoreType.DMA,
    ],
)
def _sc_aggregate(pre_hbm, src_hbm, dst_hbm, ew_hbm, out_hbm,
                  src_a, dst_a, ew_a, src_b, dst_b, ew_b,
                  rows_a, rows_b, acc, sem_ia, sem_ib, sem_g):
    c = lax.axis_index("c")
    s = lax.axis_index("s")
    row0 = s * ROWS_PER_TILE

    def _idx_copies(i, sv, dv, wv, sem):
        g = s + i * NS
        e0 = c * EDGES_PER_CORE + g * CHUNK
        return (
            pltpu.make_async_copy(src_hbm.at[pl.ds(e0, CHUNK)], sv, sem),
            pltpu.make_async_copy(dst_hbm.at[pl.ds(e0, CHUNK)], dv, sem),
            pltpu.make_async_copy(ew_hbm.at[pl.ds(e0, CHUNK)], wv, sem),
        )

    def _start_idx(i, sv, dv, wv, sem):
        @pl.when(s + i * NS < NUM_CHUNKS)
        def _():
            for cp in _idx_copies(i, sv, dv, wv, sem):
                cp.start()

    def _scale(wv, rv):
        def body(eg, carry2):
            w16 = wv[pl.ds(eg * 16, 16)]
            for k in range(16):
                e = eg * 16 + k
                wk = w16[k]
                for j in range(DOUT // 16):
                    sl = pl.ds(j * 16, 16)
                    rv[e, sl] = rv[e, sl] * wk
            return carry2

        lax.fori_loop(0, CHUNK // 16, body, 0)

    def _zero_row(r, carry):
        for j in range(DOUT // 16):
            rows_a[r, pl.ds(j * 16, 16)] = jnp.zeros((16,), jnp.float32)
        return carry

    lax.fori_loop(0, RB, _zero_row, 0)
    for b in range(ROWS_PER_TILE // RB):
        pltpu.sync_copy(rows_a.at[pl.ds(0, RB)],
                        acc.at[pl.ds(row0 + b * RB, RB)])
    plsc.subcore_barrier()

    _start_idx(0, src_a, dst_a, ew_a, sem_ia)
    _start_idx(1, src_b, dst_b, ew_b, sem_ib)

    @pl.when(s < NUM_CHUNKS)
    def _():
        for cp in _idx_copies(0, src_a, dst_a, ew_a, sem_ia):
            cp.wait()
        pltpu.make_async_copy(pre_hbm.at[src_a], rows_a, sem_g).start()

    def _slot(i, sv, dv, wv, sem, rv, nsv, ndv, nwv, nsem, nrv):
        @pl.when(s + i * NS < NUM_CHUNKS)
        def _():
            pltpu.make_async_copy(pre_hbm.at[sv], rv, sem_g).wait()

            @pl.when(s + (i + 1) * NS < NUM_CHUNKS)
            def _():
                for cp in _idx_copies(i + 1, nsv, ndv, nwv, nsem):
                    cp.wait()
                pltpu.make_async_copy(pre_hbm.at[nsv], nrv, sem_g).start()

            _scale(wv, rv)
            pltpu.sync_copy(rv, acc.at[dv], add=True)
            _start_idx(i + 2, sv, dv, wv, sem)

    def _pair(t, carry):
        _slot(2 * t, src_a, dst_a, ew_a, sem_ia, rows_a,
              src_b, dst_b, ew_b, sem_ib, rows_b)
        _slot(2 * t + 1, src_b, dst_b, ew_b, sem_ib, rows_b,
              src_a, dst_a, ew_a, sem_ia, rows_a)
        return carry

    lax.fori_loop(0, (CHUNKS_PER_TILE + 1) // 2, _pair, 0)
    plsc.subcore_barrier()

    pltpu.sync_copy(acc.at[pl.ds(row0, ROWS_PER_TILE)],
                    out_hbm.at[c, pl.ds(row0, ROWS_PER_TILE)])


def kernel(x, edge_index, edge_weight, W):
    pre = _matmul(x, W)
    partials = _sc_aggregate(pre, edge_index[0], edge_index[1], edge_weight)
    return _combine_relu(partials)
